# baseline (speedup 1.0000x reference)
import jax
import jax.numpy as jnp
from jax import lax
from jax.experimental import pallas as pl
from jax.experimental.pallas import tpu as pltpu

B, C, H, W = 8, 192, 24, 24
E, TOPK = 8, 2
HW = H * W


def _prep_kernel(a_ref, b_ref, wpre_ref, bpre_ref,
                 fc0w_ref, fc0b_ref, fc1w_ref, fc1b_ref, wt_ref,
                 xt_ref, idx_ref, cof_ref, wtb_ref):
    wtb_ref[0] = wt_ref[0].astype(jnp.bfloat16)
    wpa = wpre_ref[:, :C]
    wpb = wpre_ref[:, C:]
    dn = (((1,), (1,)), ((), ()))
    x = lax.dot_general(a_ref[0], wpa, dn,
                        preferred_element_type=jnp.float32)
    x = x + lax.dot_general(b_ref[0], wpb, dn,
                            preferred_element_type=jnp.float32)
    x = x + bpre_ref[...]
    x = jnp.where(x >= 0, x, 0.01 * x)
    xt_ref[0] = x.astype(jnp.bfloat16)

    pooled = (jnp.max(x, axis=0, keepdims=True)
              + jnp.mean(x, axis=0, keepdims=True))
    g = lax.dot_general(pooled, fc1w_ref[...], dn,
                        preferred_element_type=jnp.float32) + fc1b_ref[...]
    g = jnp.where(g >= 0, g, 0.2 * g)
    z = lax.dot_general(pooled, fc0w_ref[...], dn,
                        preferred_element_type=jnp.float32) + fc0b_ref[...]
    noise = jnp.maximum(z, 0.0) + jnp.log1p(jnp.exp(-jnp.abs(z)))
    nmean = jnp.mean(noise)
    nstd = jnp.sqrt(jnp.sum((noise - nmean) ** 2) / (E - 1))
    nstd = jnp.where(nstd == 0, 1.0, nstd)
    t = g + (noise - nmean) / nstd

    iota = lax.broadcasted_iota(jnp.int32, (1, E), 1)
    m1 = jnp.max(t)
    i1 = jnp.min(jnp.where(t == m1, iota, E))
    t2 = jnp.where(iota == i1, -jnp.float32(1e30), t)
    m2 = jnp.max(t2)
    i2 = jnp.min(jnp.where(t2 == m2, iota, E))
    g1 = jnp.sum(jnp.where(iota == i1, g, 0.0))
    g2 = jnp.sum(jnp.where(iota == i2, g, 0.0))
    mm = jnp.maximum(g1, g2)
    e1 = jnp.exp(g1 - mm)
    e2 = jnp.exp(g2 - mm)
    zs = e1 + e2
    lane2 = lax.broadcasted_iota(jnp.int32, (1, 1, TOPK), 2)
    idx_ref[...] = jnp.where(lane2 == 0, i1, i2)
    cof_ref[...] = jnp.where(lane2 == 0, e1 / zs, e2 / zs)


def _expert_kernel(idx_sref, cof_sref, xt_ref, w0_ref, w1_ref, bias_ref,
                   out_ref):
    i = pl.program_id(0)
    x = xt_ref[0]
    p_in_row = lax.broadcasted_iota(jnp.int32, (HW, C), 0) % W
    dn = (((1,), (1,)), ((), ()))

    shifts = []
    for ky in range(3):
        for kx in range(3):
            s = (ky - 1) * W + (kx - 1)
            if s > 0:
                sh = jnp.concatenate(
                    [x[s:], jnp.zeros((s, C), jnp.bfloat16)], axis=0)
            elif s < 0:
                sh = jnp.concatenate(
                    [jnp.zeros((-s, C), jnp.bfloat16), x[:s]], axis=0)
            else:
                sh = x
            if kx == 0:
                sh = jnp.where(p_in_row == 0, jnp.bfloat16(0), sh)
            elif kx == 2:
                sh = jnp.where(p_in_row == W - 1, jnp.bfloat16(0), sh)
            shifts.append(sh)

    y = jnp.zeros((HW, C), jnp.float32)
    for slot, w_ref in ((0, w0_ref), (1, w1_ref)):
        acc = jnp.zeros((HW, C), jnp.float32)
        for k in range(9):
            wk = w_ref[0, k]
            acc = acc + lax.dot_general(shifts[k], wk, dn,
                                        preferred_element_type=jnp.float32)
        e = idx_sref[i, 0, slot]
        acc = acc + bias_ref[pl.ds(e, 1), :]
        y = y + cof_sref[i, 0, slot] * acc
    out_ref[0] = y


def kernel(a, b, w_pre, b_pre, fc0_w, fc0_b, fc1_w, fc1_b,
           expert_w, expert_b):
    at = a.transpose(0, 2, 3, 1).reshape(B, HW, C)
    bt = b.transpose(0, 2, 3, 1).reshape(B, HW, C)
    wt = expert_w.transpose(0, 3, 4, 1, 2).reshape(E, 9, C, C)
    bpre = b_pre.reshape(1, C)
    fc0b = fc0_b.reshape(1, E)
    fc1b = fc1_b.reshape(1, E)

    xt, idx, cof, wtb = pl.pallas_call(
        _prep_kernel,
        grid=(B,),
        in_specs=[
            pl.BlockSpec((1, HW, C), lambda i: (i, 0, 0)),
            pl.BlockSpec((1, HW, C), lambda i: (i, 0, 0)),
            pl.BlockSpec((C, 2 * C), lambda i: (0, 0)),
            pl.BlockSpec((1, C), lambda i: (0, 0)),
            pl.BlockSpec((E, C), lambda i: (0, 0)),
            pl.BlockSpec((1, E), lambda i: (0, 0)),
            pl.BlockSpec((E, C), lambda i: (0, 0)),
            pl.BlockSpec((1, E), lambda i: (0, 0)),
            pl.BlockSpec((1, 9, C, C), lambda i: (i, 0, 0, 0)),
        ],
        out_specs=[
            pl.BlockSpec((1, HW, C), lambda i: (i, 0, 0)),
            pl.BlockSpec((1, 1, TOPK), lambda i: (i, 0, 0)),
            pl.BlockSpec((1, 1, TOPK), lambda i: (i, 0, 0)),
            pl.BlockSpec((1, 9, C, C), lambda i: (i, 0, 0, 0)),
        ],
        out_shape=[
            jax.ShapeDtypeStruct((B, HW, C), jnp.bfloat16),
            jax.ShapeDtypeStruct((B, 1, TOPK), jnp.int32),
            jax.ShapeDtypeStruct((B, 1, TOPK), jnp.float32),
            jax.ShapeDtypeStruct((E, 9, C, C), jnp.bfloat16),
        ],
    )(at, bt, w_pre, bpre, fc0_w, fc0b, fc1_w, fc1b, wt)

    grid_spec = pltpu.PrefetchScalarGridSpec(
        num_scalar_prefetch=2,
        grid=(B,),
        in_specs=[
            pl.BlockSpec((1, HW, C), lambda i, idx_s, cof_s: (i, 0, 0)),
            pl.BlockSpec((1, 9, C, C),
                         lambda i, idx_s, cof_s: (idx_s[i, 0, 0], 0, 0, 0)),
            pl.BlockSpec((1, 9, C, C),
                         lambda i, idx_s, cof_s: (idx_s[i, 0, 1], 0, 0, 0)),
            pl.BlockSpec((E, C), lambda i, idx_s, cof_s: (0, 0)),
        ],
        out_specs=pl.BlockSpec((1, HW, C),
                               lambda i, idx_s, cof_s: (i, 0, 0)),
    )
    out_t = pl.pallas_call(
        _expert_kernel,
        grid_spec=grid_spec,
        out_shape=jax.ShapeDtypeStruct((B, HW, C), jnp.float32),
    )(idx, cof, xt, wtb, wtb, expert_b)

    return out_t.reshape(B, H, W, C).transpose(0, 3, 1, 2)


# speedup vs baseline: 1.0520x; 1.0520x over previous
import jax
import jax.numpy as jnp
from jax import lax
from jax.experimental import pallas as pl
from jax.experimental.pallas import tpu as pltpu

B, C, H, W = 8, 192, 24, 24
E, TOPK = 8, 2
HW = H * W


def _prep_kernel(a_ref, b_ref, wpre_ref, bpre_ref,
                 fc0w_ref, fc0b_ref, fc1w_ref, fc1b_ref,
                 xt_ref, idx_ref, cof_ref):
    wpa = wpre_ref[:, :C]
    wpb = wpre_ref[:, C:]
    dn = (((1,), (1,)), ((), ()))
    x = lax.dot_general(a_ref[0], wpa, dn,
                        preferred_element_type=jnp.float32)
    x = x + lax.dot_general(b_ref[0], wpb, dn,
                            preferred_element_type=jnp.float32)
    x = x + bpre_ref[...]
    x = jnp.where(x >= 0, x, 0.01 * x)
    xt_ref[0] = x.astype(jnp.bfloat16)

    pooled = (jnp.max(x, axis=0, keepdims=True)
              + jnp.mean(x, axis=0, keepdims=True))
    g = lax.dot_general(pooled, fc1w_ref[...], dn,
                        preferred_element_type=jnp.float32) + fc1b_ref[...]
    g = jnp.where(g >= 0, g, 0.2 * g)
    z = lax.dot_general(pooled, fc0w_ref[...], dn,
                        preferred_element_type=jnp.float32) + fc0b_ref[...]
    noise = jnp.maximum(z, 0.0) + jnp.log1p(jnp.exp(-jnp.abs(z)))
    nmean = jnp.mean(noise)
    nstd = jnp.sqrt(jnp.sum((noise - nmean) ** 2) / (E - 1))
    nstd = jnp.where(nstd == 0, 1.0, nstd)
    t = g + (noise - nmean) / nstd

    iota = lax.broadcasted_iota(jnp.int32, (1, E), 1)
    m1 = jnp.max(t)
    i1 = jnp.min(jnp.where(t == m1, iota, E))
    t2 = jnp.where(iota == i1, -jnp.float32(1e30), t)
    m2 = jnp.max(t2)
    i2 = jnp.min(jnp.where(t2 == m2, iota, E))
    g1 = jnp.sum(jnp.where(iota == i1, g, 0.0))
    g2 = jnp.sum(jnp.where(iota == i2, g, 0.0))
    mm = jnp.maximum(g1, g2)
    e1 = jnp.exp(g1 - mm)
    e2 = jnp.exp(g2 - mm)
    zs = e1 + e2
    lane2 = lax.broadcasted_iota(jnp.int32, (1, 1, TOPK), 2)
    idx_ref[...] = jnp.where(lane2 == 0, i1, i2)
    cof_ref[...] = jnp.where(lane2 == 0, e1 / zs, e2 / zs)


def _expert_kernel(idx_sref, cof_sref, xt_ref, w0_ref, w1_ref, bias_ref,
                   out_ref):
    i = pl.program_id(0)
    x = xt_ref[0]
    p_in_row = lax.broadcasted_iota(jnp.int32, (HW, C), 0) % W
    dn = (((1,), (1,)), ((), ()))

    shifts = []
    for ky in range(3):
        for kx in range(3):
            s = (ky - 1) * W + (kx - 1)
            if s > 0:
                sh = jnp.concatenate(
                    [x[s:], jnp.zeros((s, C), jnp.bfloat16)], axis=0)
            elif s < 0:
                sh = jnp.concatenate(
                    [jnp.zeros((-s, C), jnp.bfloat16), x[:s]], axis=0)
            else:
                sh = x
            if kx == 0:
                sh = jnp.where(p_in_row == 0, jnp.bfloat16(0), sh)
            elif kx == 2:
                sh = jnp.where(p_in_row == W - 1, jnp.bfloat16(0), sh)
            shifts.append(sh)

    y = jnp.zeros((HW, C), jnp.float32)
    for slot, w_ref in ((0, w0_ref), (1, w1_ref)):
        acc = jnp.zeros((HW, C), jnp.float32)
        for k in range(9):
            wk = w_ref[0, k].astype(jnp.bfloat16)
            acc = acc + lax.dot_general(shifts[k], wk, dn,
                                        preferred_element_type=jnp.float32)
        e = idx_sref[i, 0, slot]
        acc = acc + bias_ref[pl.ds(e, 1), :]
        y = y + cof_sref[i, 0, slot] * acc
    out_ref[0] = y


def kernel(a, b, w_pre, b_pre, fc0_w, fc0_b, fc1_w, fc1_b,
           expert_w, expert_b):
    at = a.transpose(0, 2, 3, 1).reshape(B, HW, C)
    bt = b.transpose(0, 2, 3, 1).reshape(B, HW, C)
    wt = expert_w.transpose(0, 3, 4, 1, 2).reshape(E, 9, C, C)
    bpre = b_pre.reshape(1, C)
    fc0b = fc0_b.reshape(1, E)
    fc1b = fc1_b.reshape(1, E)

    xt, idx, cof = pl.pallas_call(
        _prep_kernel,
        grid=(B,),
        in_specs=[
            pl.BlockSpec((1, HW, C), lambda i: (i, 0, 0)),
            pl.BlockSpec((1, HW, C), lambda i: (i, 0, 0)),
            pl.BlockSpec((C, 2 * C), lambda i: (0, 0)),
            pl.BlockSpec((1, C), lambda i: (0, 0)),
            pl.BlockSpec((E, C), lambda i: (0, 0)),
            pl.BlockSpec((1, E), lambda i: (0, 0)),
            pl.BlockSpec((E, C), lambda i: (0, 0)),
            pl.BlockSpec((1, E), lambda i: (0, 0)),
        ],
        out_specs=[
            pl.BlockSpec((1, HW, C), lambda i: (i, 0, 0)),
            pl.BlockSpec((1, 1, TOPK), lambda i: (i, 0, 0)),
            pl.BlockSpec((1, 1, TOPK), lambda i: (i, 0, 0)),
        ],
        out_shape=[
            jax.ShapeDtypeStruct((B, HW, C), jnp.bfloat16),
            jax.ShapeDtypeStruct((B, 1, TOPK), jnp.int32),
            jax.ShapeDtypeStruct((B, 1, TOPK), jnp.float32),
        ],
    )(at, bt, w_pre, bpre, fc0_w, fc0b, fc1_w, fc1b)

    grid_spec = pltpu.PrefetchScalarGridSpec(
        num_scalar_prefetch=2,
        grid=(B,),
        in_specs=[
            pl.BlockSpec((1, HW, C), lambda i, idx_s, cof_s: (i, 0, 0)),
            pl.BlockSpec((1, 9, C, C),
                         lambda i, idx_s, cof_s: (idx_s[i, 0, 0], 0, 0, 0)),
            pl.BlockSpec((1, 9, C, C),
                         lambda i, idx_s, cof_s: (idx_s[i, 0, 1], 0, 0, 0)),
            pl.BlockSpec((E, C), lambda i, idx_s, cof_s: (0, 0)),
        ],
        out_specs=pl.BlockSpec((1, HW, C),
                               lambda i, idx_s, cof_s: (i, 0, 0)),
    )
    out_t = pl.pallas_call(
        _expert_kernel,
        grid_spec=grid_spec,
        out_shape=jax.ShapeDtypeStruct((B, HW, C), jnp.float32),
    )(idx, cof, xt, wt, wt, expert_b)

    return out_t.reshape(B, H, W, C).transpose(0, 3, 1, 2)
